# Initial kernel scaffold; baseline (speedup 1.0000x reference)
import jax
import jax.numpy as jnp
from jax.experimental import pallas as pl


def kernel(x, edge_index, edge_attr, W1, b1, W2, b2, W3, b3, W4, b4, W5, b5, W6, b6, fc_W, fc_b):
    raise NotImplementedError("write your pallas kernel here")


# speedup vs baseline: 4.3381x; 4.3381x over previous
import dataclasses
import functools

import jax
import jax.numpy as jnp
from jax import lax
from jax.experimental import pallas as pl
from jax.experimental.pallas import tpu as pltpu
from jax.experimental.pallas import tpu_sc as plsc

N = 10000
E = 320000
B = 128
NBP = 2560
EP = NBP * B
PAD = EP - E
NP = 10240
NR = NP // 16
W = 128
G = 8

_MESH = plsc.VectorSubcoreMesh(
    core_axis_name="c", subcore_axis_name="s", num_cores=2, num_subcores=16)

_SC_PARAMS = pltpu.CompilerParams()
if "needs_layout_passes" in pltpu.CompilerParams.__dataclass_fields__:
    _SC_PARAMS = dataclasses.replace(_SC_PARAMS, needs_layout_passes=False)

_DIMS = [(128, 64), (64, 128), (128, 256), (256, 512), (512, 256), (256, 128)]
_NCHUNK = {64: 1, 128: 1, 256: 2, 512: 4}


@functools.partial(
    pl.kernel,
    out_type=jax.ShapeDtypeStruct((2 * NP,), jnp.float32),
    mesh=_MESH,
    compiler_params=_SC_PARAMS,
    scratch_types=[
        pltpu.VMEM((G, B), jnp.int32),
        pltpu.VMEM((G, B), jnp.float32),
        pltpu.VMEM_SHARED((NP,), jnp.float32),
    ],
)
def _deg_kernel(src_hbm, ea_hbm, z_hbm, out_hbm, sgrp, egrp, accd):
    core = lax.axis_index("c")
    sub = lax.axis_index("s")
    row0 = sub * NR
    pltpu.sync_copy(z_hbm.at[pl.ds(row0, NR)], accd.at[pl.ds(row0, NR)])
    plsc.subcore_barrier()
    b_base = core * (NBP // 2) + sub * (NBP // 32)

    @pl.loop(0, NBP // 32 // G)
    def _(g):
        b0 = b_base + g * G
        pltpu.sync_copy(src_hbm.at[pl.ds(b0, G)], sgrp)
        pltpu.sync_copy(ea_hbm.at[pl.ds(b0, G)], egrp)
        for j in range(G):
            pltpu.sync_copy(egrp.at[j], accd.at[sgrp.at[j]], add=True)

    plsc.subcore_barrier()
    pltpu.sync_copy(accd.at[pl.ds(row0, NR)],
                    out_hbm.at[pl.ds(core * NP + row0, NR)])


@functools.partial(
    pl.kernel,
    out_type=jax.ShapeDtypeStruct((NBP, B), jnp.float32),
    mesh=_MESH,
    compiler_params=_SC_PARAMS,
    scratch_types=[
        pltpu.VMEM((NP,), jnp.float32),
        pltpu.VMEM((G, B), jnp.int32),
        pltpu.VMEM((G, B), jnp.int32),
        pltpu.VMEM((G, B), jnp.float32),
        pltpu.VMEM((G, B), jnp.float32),
    ],
)
def _norm_kernel(dis_hbm, src_hbm, dst_hbm, ea_hbm, out_hbm,
                 disb, sgrp, dgrp, egrp, ogrp):
    core = lax.axis_index("c")
    sub = lax.axis_index("s")
    wid = sub * 2 + core
    pltpu.sync_copy(dis_hbm, disb)
    b_base = wid * (NBP // 32)

    @pl.loop(0, NBP // 32 // G)
    def _(g):
        b0 = b_base + g * G
        pltpu.sync_copy(src_hbm.at[pl.ds(b0, G)], sgrp)
        pltpu.sync_copy(dst_hbm.at[pl.ds(b0, G)], dgrp)
        pltpu.sync_copy(ea_hbm.at[pl.ds(b0, G)], egrp)
        for j in range(G):
            for v in range(B // 16):
                sl = pl.ds(v * 16, 16)
                dsv = plsc.load_gather(disb, [sgrp[j, sl]])
                ddv = plsc.load_gather(disb, [dgrp[j, sl]])
                ogrp[j, sl] = -(egrp[j, sl] * dsv) * ddv
        pltpu.sync_copy(ogrp, out_hbm.at[pl.ds(b0, G)])


def _prop_body_common(h_ref, out_row_ref, acc, sgrp, dgrp, ngrp, rows,
                      src_hbm, dst_hbm, nrm_hbm, z_hbm,
                      row0, b_base, n_groups):
    pltpu.sync_copy(z_hbm.at[pl.ds(row0, NR)], acc.at[pl.ds(row0, NR)])
    plsc.subcore_barrier()

    @pl.loop(0, n_groups)
    def _(g):
        b0 = b_base + g * G
        pltpu.sync_copy(src_hbm.at[pl.ds(b0, G)], sgrp)
        pltpu.sync_copy(dst_hbm.at[pl.ds(b0, G)], dgrp)
        pltpu.sync_copy(nrm_hbm.at[pl.ds(b0, G)], ngrp)
        for j in range(G):
            pltpu.sync_copy(h_ref.at[sgrp.at[j]], rows)

            @pl.loop(0, B)
            def _(e):
                nv = plsc.load_gather(
                    ngrp.at[j], [jnp.full((16,), e, jnp.int32)])
                for w in range(W // 16):
                    sl = pl.ds(w * 16, 16)
                    rows[e, sl] = rows[e, sl] * nv

            pltpu.sync_copy(rows, acc.at[dgrp.at[j]], add=True)

    plsc.subcore_barrier()
    pltpu.sync_copy(acc.at[pl.ds(row0, NR)], out_row_ref.at[pl.ds(row0, NR)])
    plsc.subcore_barrier()


_PROP_SCRATCH = [
    pltpu.VMEM((G, B), jnp.int32),
    pltpu.VMEM((G, B), jnp.int32),
    pltpu.VMEM((G, B), jnp.float32),
    pltpu.VMEM((B, W), jnp.float32),
    pltpu.VMEM_SHARED((NP, W), jnp.float32),
]


@functools.partial(
    pl.kernel,
    out_type=[jax.ShapeDtypeStruct((NP, W), jnp.float32) for _ in range(2)],
    mesh=_MESH,
    compiler_params=_SC_PARAMS,
    scratch_types=_PROP_SCRATCH,
)
def _prop_es(h_hbm, src_hbm, dst_hbm, nrm_hbm, z_hbm, o0_hbm, o1_hbm,
             sgrp, dgrp, ngrp, rows, acc):
    core = lax.axis_index("c")
    sub = lax.axis_index("s")
    row0 = sub * NR
    per_core = NBP // 2
    per_tile = per_core // 16
    outs = (o0_hbm, o1_hbm)
    for k in range(2):
        @pl.when(core == k)
        def _():
            b_base = k * per_core + sub * per_tile
            _prop_body_common(h_hbm, outs[k], acc, sgrp, dgrp, ngrp, rows,
                              src_hbm, dst_hbm, nrm_hbm, z_hbm,
                              row0, b_base, per_tile // G)


def _make_prop_fs(Q):
    nchunks = 2 * Q
    out_type = [jax.ShapeDtypeStruct((NP, W), jnp.float32)
                for _ in range(nchunks)]

    @functools.partial(pl.kernel, out_type=out_type, mesh=_MESH,
                       compiler_params=_SC_PARAMS,
                       scratch_types=_PROP_SCRATCH)
    def prop(*refs):
        h_refs = refs[:nchunks]
        src_hbm, dst_hbm, nrm_hbm, z_hbm = refs[nchunks:nchunks + 4]
        out_refs = refs[nchunks + 4:nchunks + 4 + nchunks]
        sgrp, dgrp, ngrp, rows, acc = refs[nchunks + 4 + nchunks:]
        core = lax.axis_index("c")
        sub = lax.axis_index("s")
        row0 = sub * NR
        per_tile = NBP // 16
        b_base = sub * per_tile
        for k in range(2):
            @pl.when(core == k)
            def _():
                for q in range(Q):
                    ch = k * Q + q
                    _prop_body_common(h_refs[ch], out_refs[ch], acc,
                                      sgrp, dgrp, ngrp, rows,
                                      src_hbm, dst_hbm, nrm_hbm, z_hbm,
                                      row0, b_base, per_tile // G)

    return prop


_PROP_FS = {2: _make_prop_fs(1), 4: _make_prop_fs(2)}


def _dis_body(degp_ref, out_ref):
    d = degp_ref[0] + degp_ref[1]
    pos = d > 0.0
    out_ref[...] = jnp.where(pos, 1.0 / jnp.sqrt(jnp.where(pos, d, 1.0)), 0.0)


def _dis_call(degp):
    out = pl.pallas_call(
        _dis_body,
        out_shape=jax.ShapeDtypeStruct((NP // 128, 128), jnp.float32),
    )(degp.reshape(2, NP // 128, 128))
    return out.reshape(NP)


def _add_body(a_ref, b_ref, out_ref):
    out_ref[...] = a_ref[...] + b_ref[...]


_R = 1024


def _combine_call(a, b):
    spec = pl.BlockSpec((_R, W), lambda i: (i, 0))
    return pl.pallas_call(
        _add_body, grid=(NP // _R,), in_specs=[spec, spec], out_specs=spec,
        out_shape=jax.ShapeDtypeStruct((NP, W), jnp.float32),
    )(a, b)


def _make_layer(li):
    fin, fout = _DIMS[li]
    nin = _NCHUNK[fin]
    es = nin == 1
    finp = nin * W
    if li == 5:
        nout = 1
    else:
        nout = _NCHUNK[_DIMS[li + 1][0]]
    foutp = nout * W

    n_t1 = 1 if es else nin
    n_p2 = 2 if es else nin

    def body(*refs):
        h = refs[:nin]
        t1 = refs[nin:nin + n_t1]
        p2 = refs[nin + n_t1:nin + n_t1 + n_p2]
        w02, w1, w2x2, bias = refs[nin + n_t1 + n_p2:nin + n_t1 + n_p2 + 4]
        outs = refs[nin + n_t1 + n_p2 + 4:]
        acc = jnp.zeros((_R, foutp), jnp.float32)
        if es:
            p2s = p2[0][...] + p2[1][...]
            acc += jnp.dot(h[0][...], w02[0],
                           preferred_element_type=jnp.float32)
            acc += jnp.dot(t1[0][...], w1[0],
                           preferred_element_type=jnp.float32)
            acc += jnp.dot(p2s, w2x2[0], preferred_element_type=jnp.float32)
        else:
            for c in range(nin):
                acc += jnp.dot(h[c][...], w02[c],
                               preferred_element_type=jnp.float32)
                acc += jnp.dot(t1[c][...], w1[c],
                               preferred_element_type=jnp.float32)
                acc += jnp.dot(p2[c][...], w2x2[c],
                               preferred_element_type=jnp.float32)
        acc = jnp.maximum(acc + bias[...], 0.0)
        for co in range(nout):
            outs[co][...] = acc[:, co * W:(co + 1) * W]

    chunk_spec = pl.BlockSpec((_R, W), lambda i: (i, 0))
    w_spec = pl.BlockSpec((nin, W, foutp), lambda i: (0, 0, 0))
    in_specs = ([chunk_spec] * (nin + n_t1 + n_p2)
                + [w_spec] * 3
                + [pl.BlockSpec((1, foutp), lambda i: (0, 0))])
    out_specs = [chunk_spec for _ in range(nout)]
    out_shape = [jax.ShapeDtypeStruct((NP, W), jnp.float32)
                 for _ in range(nout)]

    call = pl.pallas_call(
        body, grid=(NP // _R,), in_specs=in_specs, out_specs=out_specs,
        out_shape=out_shape)

    def _pad_w(m):
        m = jnp.pad(m, ((0, finp - m.shape[0]), (0, foutp - m.shape[1])))
        return m.reshape(nin, W, foutp)

    def run(h_chunks, t1_chunks, p2_chunks, Wmat, b):
        w02 = _pad_w(Wmat[0] - Wmat[2])
        w1 = _pad_w(Wmat[1])
        w2x2 = _pad_w(2.0 * Wmat[2])
        bp = jnp.pad(b, (0, foutp - b.shape[0])).reshape(1, foutp)
        outs = call(*h_chunks, *t1_chunks, *p2_chunks, w02, w1, w2x2, bp)
        return tuple(outs)

    return run


_LAYER = [_make_layer(i) for i in range(6)]


def _fc_body(h2_ref, w_ref, b_ref, out_ref):
    logits = jnp.dot(h2_ref[...], w_ref[...],
                     preferred_element_type=jnp.float32) + b_ref[...]
    m = jnp.max(logits, axis=1, keepdims=True)
    e = jnp.exp(logits - m)
    out_ref[...] = e / jnp.sum(e, axis=1, keepdims=True)


def _fc_call(h2, fc_W, fc_b):
    return pl.pallas_call(
        _fc_body,
        out_shape=jax.ShapeDtypeStruct((100, 2), jnp.float32),
    )(h2, fc_W, fc_b.reshape(1, 2))


def kernel(x, edge_index, edge_attr, W1, b1, W2, b2, W3, b3, W4, b4,
           W5, b5, W6, b6, fc_W, fc_b):
    src = edge_index[0]
    dst = edge_index[1]
    spread = (jnp.arange(PAD, dtype=jnp.int32) * 37) % N
    srcP = jnp.concatenate([src, spread]).reshape(NBP, B)
    dstP = jnp.concatenate([dst, spread]).reshape(NBP, B)
    eaP = jnp.concatenate(
        [edge_attr, jnp.zeros((PAD,), jnp.float32)]).reshape(NBP, B)

    zN = jnp.zeros((NP,), jnp.float32)
    zNW = jnp.zeros((NP, W), jnp.float32)

    degp = _deg_kernel(srcP, eaP, zN)
    dis = _dis_call(degp)
    normP = _norm_kernel(dis, srcP, dstP, eaP)

    weights = [(W1, b1), (W2, b2), (W3, b3), (W4, b4), (W5, b5), (W6, b6)]

    xp = jnp.concatenate([x, jnp.zeros((NP - N, 128), jnp.float32)], axis=0)
    h_chunks = (xp,)
    for li in range(6):
        fin, _fout = _DIMS[li]
        nin = _NCHUNK[fin]
        if nin == 1:
            t1a, t1b = _prop_es(h_chunks[0], srcP, dstP, normP, zNW)
            t1 = _combine_call(t1a, t1b)
            p2a, p2b = _prop_es(t1, srcP, dstP, normP, zNW)
            h_chunks = _LAYER[li](h_chunks, (t1,), (p2a, p2b),
                                  weights[li][0], weights[li][1])
        else:
            prop = _PROP_FS[nin]
            t1c = prop(*h_chunks, srcP, dstP, normP, zNW)
            p2c = prop(*t1c, srcP, dstP, normP, zNW)
            h_chunks = _LAYER[li](h_chunks, tuple(t1c), tuple(p2c),
                                  weights[li][0], weights[li][1])

    h6 = h_chunks[0][:N]
    h2 = h6.reshape(100, 128 * 100)
    return _fc_call(h2, fc_W, fc_b)


# speedup vs baseline: 6.9071x; 1.5922x over previous
import dataclasses
import functools

import jax
import jax.numpy as jnp
from jax import lax
from jax.experimental import pallas as pl
from jax.experimental.pallas import tpu as pltpu
from jax.experimental.pallas import tpu_sc as plsc

N = 10000
E = 320000
B = 128
NBP = 2560
EP = NBP * B
PAD = EP - E
NP = 10240
NR = NP // 16
W = 128
G = 8

_MESH = plsc.VectorSubcoreMesh(
    core_axis_name="c", subcore_axis_name="s", num_cores=2, num_subcores=16)

_SC_PARAMS = pltpu.CompilerParams()
if "needs_layout_passes" in pltpu.CompilerParams.__dataclass_fields__:
    _SC_PARAMS = dataclasses.replace(_SC_PARAMS, needs_layout_passes=False)

_DIMS = [(128, 64), (64, 128), (128, 256), (256, 512), (512, 256), (256, 128)]
_NCHUNK = {64: 1, 128: 1, 256: 2, 512: 4}


@functools.partial(
    pl.kernel,
    out_type=jax.ShapeDtypeStruct((2 * NP,), jnp.float32),
    mesh=_MESH,
    compiler_params=_SC_PARAMS,
    scratch_types=[
        pltpu.VMEM((G, B), jnp.int32),
        pltpu.VMEM((G, B), jnp.float32),
        pltpu.VMEM_SHARED((NP,), jnp.float32),
    ],
)
def _deg_kernel(src_hbm, ea_hbm, z_hbm, out_hbm, sgrp, egrp, accd):
    core = lax.axis_index("c")
    sub = lax.axis_index("s")
    row0 = sub * NR
    pltpu.sync_copy(z_hbm.at[pl.ds(row0, NR)], accd.at[pl.ds(row0, NR)])
    plsc.subcore_barrier()
    b_base = core * (NBP // 2) + sub * (NBP // 32)

    @pl.loop(0, NBP // 32 // G)
    def _(g):
        b0 = b_base + g * G
        pltpu.sync_copy(src_hbm.at[pl.ds(b0, G)], sgrp)
        pltpu.sync_copy(ea_hbm.at[pl.ds(b0, G)], egrp)
        for j in range(G):
            pltpu.sync_copy(egrp.at[j], accd.at[sgrp.at[j]], add=True)

    plsc.subcore_barrier()
    pltpu.sync_copy(accd.at[pl.ds(row0, NR)],
                    out_hbm.at[pl.ds(core * NP + row0, NR)])


@functools.partial(
    pl.kernel,
    out_type=jax.ShapeDtypeStruct((NBP, B), jnp.float32),
    mesh=_MESH,
    compiler_params=_SC_PARAMS,
    scratch_types=[
        pltpu.VMEM((NP,), jnp.float32),
        pltpu.VMEM((G, B), jnp.int32),
        pltpu.VMEM((G, B), jnp.int32),
        pltpu.VMEM((G, B), jnp.float32),
        pltpu.VMEM((G, B), jnp.float32),
    ],
)
def _norm_kernel(dis_hbm, src_hbm, dst_hbm, ea_hbm, out_hbm,
                 disb, sgrp, dgrp, egrp, ogrp):
    core = lax.axis_index("c")
    sub = lax.axis_index("s")
    wid = sub * 2 + core
    pltpu.sync_copy(dis_hbm, disb)
    b_base = wid * (NBP // 32)

    @pl.loop(0, NBP // 32 // G)
    def _(g):
        b0 = b_base + g * G
        pltpu.sync_copy(src_hbm.at[pl.ds(b0, G)], sgrp)
        pltpu.sync_copy(dst_hbm.at[pl.ds(b0, G)], dgrp)
        pltpu.sync_copy(ea_hbm.at[pl.ds(b0, G)], egrp)
        for j in range(G):
            for v in range(B // 16):
                sl = pl.ds(v * 16, 16)
                dsv = plsc.load_gather(disb, [sgrp[j, sl]])
                ddv = plsc.load_gather(disb, [dgrp[j, sl]])
                ogrp[j, sl] = -(egrp[j, sl] * dsv) * ddv
        pltpu.sync_copy(ogrp, out_hbm.at[pl.ds(b0, G)])


DEPTH = 2


def _prop_body_common(h_ref, out_row_ref, acc, sgrp, dgrp, ngrp, rows,
                      gsem, ssem, isem,
                      src_hbm, dst_hbm, nrm_hbm, z_hbm,
                      row0, b_base, n_igroups):
    pltpu.sync_copy(z_hbm.at[pl.ds(row0, NR)], acc.at[pl.ds(row0, NR)])
    plsc.subcore_barrier()

    def issue_idx(slot, b0):
        pltpu.async_copy(src_hbm.at[pl.ds(b0, G)], sgrp.at[slot], isem)
        pltpu.async_copy(dst_hbm.at[pl.ds(b0, G)], dgrp.at[slot], isem)
        pltpu.async_copy(nrm_hbm.at[pl.ds(b0, G)], ngrp.at[slot], isem)

    def wait_idx(slot):
        pltpu.make_async_copy(src_hbm.at[pl.ds(0, G)], sgrp.at[slot],
                              isem).wait()
        pltpu.make_async_copy(dst_hbm.at[pl.ds(0, G)], dgrp.at[slot],
                              isem).wait()
        pltpu.make_async_copy(nrm_hbm.at[pl.ds(0, G)], ngrp.at[slot],
                              isem).wait()

    def half(slot, h0):
        @pl.loop(0, DEPTH)
        def _(b):
            pltpu.make_async_copy(rows.at[b], acc.at[dgrp.at[slot, h0 + b]],
                                  ssem.at[b]).wait()
            pltpu.async_copy(h_ref.at[sgrp.at[slot, h0 + b]], rows.at[b],
                             gsem.at[b])

        @pl.loop(0, DEPTH)
        def _(b):
            pltpu.make_async_copy(h_ref.at[sgrp.at[slot, h0 + b]],
                                  rows.at[b], gsem.at[b]).wait()

            @pl.loop(0, B)
            def _(e):
                nv = plsc.load_gather(
                    ngrp.at[slot, h0 + b], [jnp.full((16,), e, jnp.int32)])
                for w in range(W // 16):
                    sl = pl.ds(w * 16, 16)
                    rows[b, e, sl] = rows[b, e, sl] * nv

            pltpu.async_copy(rows.at[b], acc.at[dgrp.at[slot, h0 + b]],
                             ssem.at[b], add=True)

    @pl.loop(0, DEPTH)
    def _(b):
        pltpu.async_copy(z_hbm.at[pl.ds(0, B)], rows.at[b], ssem.at[b])

    issue_idx(0, b_base)

    @pl.loop(0, n_igroups // 2)
    def _(i):
        for S in range(2):
            gidx = 2 * i + S
            wait_idx(S)
            half(S, 0)

            @pl.when(gidx + 1 < n_igroups)
            def _():
                issue_idx(1 - S, b_base + (gidx + 1) * G)

            for h0 in range(DEPTH, G, DEPTH):
                half(S, h0)

    @pl.loop(0, DEPTH)
    def _(b):
        pltpu.make_async_copy(rows.at[b], acc.at[dgrp.at[0, b]],
                              ssem.at[b]).wait()

    plsc.subcore_barrier()
    pltpu.sync_copy(acc.at[pl.ds(row0, NR)], out_row_ref.at[pl.ds(row0, NR)])
    plsc.subcore_barrier()


_PROP_SCRATCH = [
    pltpu.VMEM((2, G, B), jnp.int32),
    pltpu.VMEM((2, G, B), jnp.int32),
    pltpu.VMEM((2, G, B), jnp.float32),
    pltpu.VMEM((DEPTH, B, W), jnp.float32),
    pltpu.VMEM_SHARED((NP, W), jnp.float32),
    pltpu.SemaphoreType.DMA((DEPTH,)),
    pltpu.SemaphoreType.DMA((DEPTH,)),
    pltpu.SemaphoreType.DMA,
]


@functools.partial(
    pl.kernel,
    out_type=[jax.ShapeDtypeStruct((NP, W), jnp.float32) for _ in range(2)],
    mesh=_MESH,
    compiler_params=_SC_PARAMS,
    scratch_types=_PROP_SCRATCH,
)
def _prop_es(h_hbm, src_hbm, dst_hbm, nrm_hbm, z_hbm, o0_hbm, o1_hbm,
             sgrp, dgrp, ngrp, rows, acc, gsem, ssem, isem):
    core = lax.axis_index("c")
    sub = lax.axis_index("s")
    row0 = sub * NR
    per_core = NBP // 2
    per_tile = per_core // 16
    outs = (o0_hbm, o1_hbm)
    for k in range(2):
        @pl.when(core == k)
        def _():
            b_base = k * per_core + sub * per_tile
            _prop_body_common(h_hbm, outs[k], acc, sgrp, dgrp, ngrp, rows,
                              gsem, ssem, isem,
                              src_hbm, dst_hbm, nrm_hbm, z_hbm,
                              row0, b_base, per_tile // G)


def _make_prop_fs(Q):
    nchunks = 2 * Q
    out_type = [jax.ShapeDtypeStruct((NP, W), jnp.float32)
                for _ in range(nchunks)]

    @functools.partial(pl.kernel, out_type=out_type, mesh=_MESH,
                       compiler_params=_SC_PARAMS,
                       scratch_types=_PROP_SCRATCH)
    def prop(*refs):
        h_refs = refs[:nchunks]
        src_hbm, dst_hbm, nrm_hbm, z_hbm = refs[nchunks:nchunks + 4]
        out_refs = refs[nchunks + 4:nchunks + 4 + nchunks]
        (sgrp, dgrp, ngrp, rows, acc,
         gsem, ssem, isem) = refs[nchunks + 4 + nchunks:]
        core = lax.axis_index("c")
        sub = lax.axis_index("s")
        row0 = sub * NR
        per_tile = NBP // 16
        b_base = sub * per_tile
        for k in range(2):
            @pl.when(core == k)
            def _():
                for q in range(Q):
                    ch = k * Q + q
                    _prop_body_common(h_refs[ch], out_refs[ch], acc,
                                      sgrp, dgrp, ngrp, rows,
                                      gsem, ssem, isem,
                                      src_hbm, dst_hbm, nrm_hbm, z_hbm,
                                      row0, b_base, per_tile // G)

    return prop


_PROP_FS = {2: _make_prop_fs(1), 4: _make_prop_fs(2)}


def _dis_body(degp_ref, out_ref):
    d = degp_ref[0] + degp_ref[1]
    pos = d > 0.0
    out_ref[...] = jnp.where(pos, 1.0 / jnp.sqrt(jnp.where(pos, d, 1.0)), 0.0)


def _dis_call(degp):
    out = pl.pallas_call(
        _dis_body,
        out_shape=jax.ShapeDtypeStruct((NP // 128, 128), jnp.float32),
    )(degp.reshape(2, NP // 128, 128))
    return out.reshape(NP)


def _add_body(a_ref, b_ref, out_ref):
    out_ref[...] = a_ref[...] + b_ref[...]


_R = 1024


def _combine_call(a, b):
    spec = pl.BlockSpec((_R, W), lambda i: (i, 0))
    return pl.pallas_call(
        _add_body, grid=(NP // _R,), in_specs=[spec, spec], out_specs=spec,
        out_shape=jax.ShapeDtypeStruct((NP, W), jnp.float32),
    )(a, b)


def _make_layer(li):
    fin, fout = _DIMS[li]
    nin = _NCHUNK[fin]
    es = nin == 1
    finp = nin * W
    if li == 5:
        nout = 1
    else:
        nout = _NCHUNK[_DIMS[li + 1][0]]
    foutp = nout * W

    n_t1 = 1 if es else nin
    n_p2 = 2 if es else nin

    def body(*refs):
        h = refs[:nin]
        t1 = refs[nin:nin + n_t1]
        p2 = refs[nin + n_t1:nin + n_t1 + n_p2]
        w02, w1, w2x2, bias = refs[nin + n_t1 + n_p2:nin + n_t1 + n_p2 + 4]
        outs = refs[nin + n_t1 + n_p2 + 4:]
        acc = jnp.zeros((_R, foutp), jnp.float32)
        if es:
            p2s = p2[0][...] + p2[1][...]
            acc += jnp.dot(h[0][...], w02[0],
                           preferred_element_type=jnp.float32)
            acc += jnp.dot(t1[0][...], w1[0],
                           preferred_element_type=jnp.float32)
            acc += jnp.dot(p2s, w2x2[0], preferred_element_type=jnp.float32)
        else:
            for c in range(nin):
                acc += jnp.dot(h[c][...], w02[c],
                               preferred_element_type=jnp.float32)
                acc += jnp.dot(t1[c][...], w1[c],
                               preferred_element_type=jnp.float32)
                acc += jnp.dot(p2[c][...], w2x2[c],
                               preferred_element_type=jnp.float32)
        acc = jnp.maximum(acc + bias[...], 0.0)
        for co in range(nout):
            outs[co][...] = acc[:, co * W:(co + 1) * W]

    chunk_spec = pl.BlockSpec((_R, W), lambda i: (i, 0))
    w_spec = pl.BlockSpec((nin, W, foutp), lambda i: (0, 0, 0))
    in_specs = ([chunk_spec] * (nin + n_t1 + n_p2)
                + [w_spec] * 3
                + [pl.BlockSpec((1, foutp), lambda i: (0, 0))])
    out_specs = [chunk_spec for _ in range(nout)]
    out_shape = [jax.ShapeDtypeStruct((NP, W), jnp.float32)
                 for _ in range(nout)]

    call = pl.pallas_call(
        body, grid=(NP // _R,), in_specs=in_specs, out_specs=out_specs,
        out_shape=out_shape)

    def _pad_w(m):
        m = jnp.pad(m, ((0, finp - m.shape[0]), (0, foutp - m.shape[1])))
        return m.reshape(nin, W, foutp)

    def run(h_chunks, t1_chunks, p2_chunks, Wmat, b):
        w02 = _pad_w(Wmat[0] - Wmat[2])
        w1 = _pad_w(Wmat[1])
        w2x2 = _pad_w(2.0 * Wmat[2])
        bp = jnp.pad(b, (0, foutp - b.shape[0])).reshape(1, foutp)
        outs = call(*h_chunks, *t1_chunks, *p2_chunks, w02, w1, w2x2, bp)
        return tuple(outs)

    return run


_LAYER = [_make_layer(i) for i in range(6)]


def _fc_body(h2_ref, w_ref, b_ref, out_ref):
    logits = jnp.dot(h2_ref[...], w_ref[...],
                     preferred_element_type=jnp.float32) + b_ref[...]
    m = jnp.max(logits, axis=1, keepdims=True)
    e = jnp.exp(logits - m)
    out_ref[...] = e / jnp.sum(e, axis=1, keepdims=True)


def _fc_call(h2, fc_W, fc_b):
    return pl.pallas_call(
        _fc_body,
        out_shape=jax.ShapeDtypeStruct((100, 2), jnp.float32),
    )(h2, fc_W, fc_b.reshape(1, 2))


def kernel(x, edge_index, edge_attr, W1, b1, W2, b2, W3, b3, W4, b4,
           W5, b5, W6, b6, fc_W, fc_b):
    src = edge_index[0]
    dst = edge_index[1]
    spread = (jnp.arange(PAD, dtype=jnp.int32) * 37) % N
    srcP = jnp.concatenate([src, spread]).reshape(NBP, B)
    dstP = jnp.concatenate([dst, spread]).reshape(NBP, B)
    eaP = jnp.concatenate(
        [edge_attr, jnp.zeros((PAD,), jnp.float32)]).reshape(NBP, B)

    zN = jnp.zeros((NP,), jnp.float32)
    zNW = jnp.zeros((NP, W), jnp.float32)

    degp = _deg_kernel(srcP, eaP, zN)
    dis = _dis_call(degp)
    normP = _norm_kernel(dis, srcP, dstP, eaP)

    weights = [(W1, b1), (W2, b2), (W3, b3), (W4, b4), (W5, b5), (W6, b6)]

    xp = jnp.concatenate([x, jnp.zeros((NP - N, 128), jnp.float32)], axis=0)
    h_chunks = (xp,)
    for li in range(6):
        fin, _fout = _DIMS[li]
        nin = _NCHUNK[fin]
        if nin == 1:
            t1a, t1b = _prop_es(h_chunks[0], srcP, dstP, normP, zNW)
            t1 = _combine_call(t1a, t1b)
            p2a, p2b = _prop_es(t1, srcP, dstP, normP, zNW)
            h_chunks = _LAYER[li](h_chunks, (t1,), (p2a, p2b),
                                  weights[li][0], weights[li][1])
        else:
            prop = _PROP_FS[nin]
            t1c = prop(*h_chunks, srcP, dstP, normP, zNW)
            p2c = prop(*t1c, srcP, dstP, normP, zNW)
            h_chunks = _LAYER[li](h_chunks, tuple(t1c), tuple(p2c),
                                  weights[li][0], weights[li][1])

    h6 = h_chunks[0][:N]
    h2 = h6.reshape(100, 128 * 100)
    return _fc_call(h2, fc_W, fc_b)
